# baseline (speedup 1.0000x reference)
import jax
import jax.numpy as jnp
from jax import lax
from jax.experimental import pallas as pl
from jax.experimental.pallas import tpu as pltpu
from jax.experimental.pallas import tpu_sc as plsc

N_NODES = 10000
N_PAD = 10112
N_EDGES = 320000
D = 128
NUM_CLASSES = 64

NC = 2
NS = 16
NW = NC * NS
EPT = N_EDGES // NW
CHUNK = 128
EPT_PAD = 10112
NCH = EPT_PAD // CHUNK
ROWS_PT = N_PAD // NS

_f32 = jnp.float32


def _make_sc_agg(with_deg):
  mesh = plsc.VectorSubcoreMesh(core_axis_name="c", subcore_axis_name="s")

  out_type = [jax.ShapeDtypeStruct((NC, N_PAD, D), _f32)]
  scratch = [
      pltpu.VMEM((2 * NCH, CHUNK), jnp.int32),
      pltpu.VMEM((CHUNK, D), _f32),
      pltpu.VMEM_SHARED((N_PAD, D), _f32),
      pltpu.SemaphoreType.DMA,
  ]
  if with_deg:
    out_type.append(jax.ShapeDtypeStruct((NC * N_PAD,), _f32))
    scratch += [
        pltpu.VMEM((CHUNK,), _f32),
        pltpu.VMEM((ROWS_PT,), _f32),
        pltpu.VMEM_SHARED((N_PAD,), _f32),
    ]

  def body(table_hbm, ed_hbm, *rest):
    if with_deg:
      (zdeg_hbm, ones_hbm, agg_out, deg_out, ed_t, rows_v,
       agg_sh, sem_g, ones_v, zvec, deg_sh) = rest
    else:
      agg_out, ed_t, rows_v, agg_sh, sem_g = rest
    c = lax.axis_index("c")
    s = lax.axis_index("s")
    wid = s * NC + c
    r0 = s * ROWS_PT

    def zfill_row(i, _):
      def zfill_col(j, _):
        rows_v[i, pl.ds(j * 16, 16)] = jnp.zeros((16,), _f32)
        return 0
      return lax.fori_loop(0, D // 16, zfill_col, 0)
    lax.fori_loop(0, 8, zfill_row, 0)
    def zcopy(r, _):
      pltpu.sync_copy(rows_v.at[pl.ds(0, 8), :],
                      agg_sh.at[pl.ds(r0 + r * 8, 8), :])
      return 0
    lax.fori_loop(0, ROWS_PT // 8, zcopy, 0)
    if with_deg:
      pltpu.sync_copy(ones_hbm, ones_v)
      pltpu.sync_copy(zdeg_hbm, zvec)
      pltpu.sync_copy(zvec, deg_sh.at[pl.ds(r0, ROWS_PT)])

    plsc.subcore_barrier()

    pltpu.sync_copy(ed_hbm.at[wid], ed_t)

    def step(i, _):
      pltpu.async_copy(table_hbm.at[ed_t.at[2 * i]], rows_v, sem_g).wait()
      pltpu.sync_copy(rows_v, agg_sh.at[ed_t.at[2 * i + 1]], add=True)
      if with_deg:
        pltpu.sync_copy(ones_v, deg_sh.at[ed_t.at[2 * i + 1]], add=True)
      return 0
    lax.fori_loop(0, NCH, step, 0)

    plsc.subcore_barrier()

    pltpu.sync_copy(agg_sh.at[pl.ds(r0, ROWS_PT), :],
                    agg_out.at[c, pl.ds(r0, ROWS_PT), :])
    if with_deg:
      pltpu.sync_copy(deg_sh.at[pl.ds(r0, ROWS_PT)], zvec)
      pltpu.sync_copy(zvec, deg_out.at[pl.ds(c * N_PAD + r0, ROWS_PT)])

  return pl.kernel(body, out_type=out_type, mesh=mesh, scratch_types=scratch)


_sc_agg_deg = _make_sc_agg(True)
_sc_agg = _make_sc_agg(False)

ROWS_B = 400
GRID = N_NODES // ROWS_B


def _tc1_body(p_ref, d_ref, w_ref, b_ref, o_ref):
  a = p_ref[0] + p_ref[1]
  d = jnp.maximum(d_ref[0] + d_ref[1], 1.0)
  a = a / d
  h = jnp.dot(a, w_ref[...], preferred_element_type=_f32) + b_ref[...]
  o_ref[...] = jnp.maximum(h, 0.0)


_tc1 = pl.pallas_call(
    _tc1_body,
    grid=(GRID,),
    in_specs=[
        pl.BlockSpec((NC, ROWS_B, D), lambda j: (0, j, 0)),
        pl.BlockSpec((NC, ROWS_B, 1), lambda j: (0, j, 0)),
        pl.BlockSpec((D, D), lambda j: (0, 0)),
        pl.BlockSpec((1, D), lambda j: (0, 0)),
    ],
    out_specs=pl.BlockSpec((ROWS_B, D), lambda j: (j, 0)),
    out_shape=jax.ShapeDtypeStruct((N_NODES, D), _f32),
)


def _tc2_body(p_ref, d_ref, w2_ref, b2_ref, wd_ref, bd_ref, o_ref):
  a = p_ref[0] + p_ref[1]
  d = jnp.maximum(d_ref[0] + d_ref[1], 1.0)
  a = a / d
  h = jnp.maximum(
      jnp.dot(a, w2_ref[...], preferred_element_type=_f32) + b2_ref[...], 0.0)
  lg = jnp.dot(h, wd_ref[...], preferred_element_type=_f32) + bd_ref[...]
  m = jnp.max(lg, axis=-1, keepdims=True)
  e = jnp.exp(lg - m)
  o_ref[...] = e / jnp.sum(e, axis=-1, keepdims=True)


_tc2 = pl.pallas_call(
    _tc2_body,
    grid=(GRID,),
    in_specs=[
        pl.BlockSpec((NC, ROWS_B, D), lambda j: (0, j, 0)),
        pl.BlockSpec((NC, ROWS_B, 1), lambda j: (0, j, 0)),
        pl.BlockSpec((D, D), lambda j: (0, 0)),
        pl.BlockSpec((1, D), lambda j: (0, 0)),
        pl.BlockSpec((D, NUM_CLASSES), lambda j: (0, 0)),
        pl.BlockSpec((1, NUM_CLASSES), lambda j: (0, 0)),
    ],
    out_specs=pl.BlockSpec((ROWS_B, NUM_CLASSES), lambda j: (j, 0)),
    out_shape=jax.ShapeDtypeStruct((N_NODES, NUM_CLASSES), _f32),
)


def kernel(x, edge_index, W1, b1, W2, b2, Wd, bd):
  src = jnp.pad(edge_index[0].reshape(NW, EPT), ((0, 0), (0, EPT_PAD - EPT)))
  dst = jnp.pad(edge_index[1].reshape(NW, EPT), ((0, 0), (0, EPT_PAD - EPT)),
                constant_values=N_NODES)
  ed = jnp.stack([src.reshape(NW, NCH, CHUNK), dst.reshape(NW, NCH, CHUNK)],
                 axis=2).reshape(NW, 2 * NCH, CHUNK)
  zdeg = jnp.zeros((ROWS_PT,), _f32)
  ones1 = jnp.ones((CHUNK,), _f32)
  agg1p, degf = _sc_agg_deg(x, ed, zdeg, ones1)
  degp = degf.reshape(NC, N_PAD, 1)
  h1 = _tc1(agg1p, degp, W1, b1.reshape(1, D))
  agg2p, = _sc_agg(h1, ed)
  out = _tc2(agg2p, degp, W2, b2.reshape(1, D), Wd, bd.reshape(1, NUM_CLASSES))
  return out


# speedup vs baseline: 1.9495x; 1.9495x over previous
import jax
import jax.numpy as jnp
from jax import lax
from jax.experimental import pallas as pl
from jax.experimental.pallas import tpu as pltpu
from jax.experimental.pallas import tpu_sc as plsc

N_NODES = 10000
N_PAD = 10112
N_EDGES = 320000
D = 128
NUM_CLASSES = 64

NC = 2
NS = 16
NW = NC * NS
EPT = N_EDGES // NW
CHUNK = 125
NCH = EPT // CHUNK
NPASS = NCH // 2
ROWS_PT = N_PAD // NS

_f32 = jnp.float32


def _make_sc_agg(with_deg):
  mesh = plsc.VectorSubcoreMesh(core_axis_name="c", subcore_axis_name="s")

  out_type = [jax.ShapeDtypeStruct((NC, N_PAD, D), _f32)]
  scratch = [
      pltpu.VMEM((NPASS, CHUNK), jnp.int32),
      pltpu.VMEM((NPASS, CHUNK), jnp.int32),
      pltpu.VMEM((2, CHUNK, D), _f32),
      pltpu.VMEM_SHARED((N_PAD, D), _f32),
      pltpu.SemaphoreType.DMA,
  ]
  if with_deg:
    out_type.append(jax.ShapeDtypeStruct((NC * N_PAD,), _f32))
    scratch += [
        pltpu.VMEM((CHUNK,), _f32),
        pltpu.VMEM((ROWS_PT,), _f32),
        pltpu.VMEM_SHARED((N_PAD,), _f32),
    ]

  def body(table_hbm, src_hbm, dst_hbm, *rest):
    if with_deg:
      (zdeg_hbm, ones_hbm, agg_out, deg_out, src_t, dst_t, rows_v,
       agg_sh, sem_g, ones_v, zvec, deg_sh) = rest
    else:
      agg_out, src_t, dst_t, rows_v, agg_sh, sem_g = rest
    c = lax.axis_index("c")
    s = lax.axis_index("s")
    wid = s * NC + c
    r0 = s * ROWS_PT

    def zfill_row(i, _):
      def zfill_col(j, _):
        rows_v[0, i, pl.ds(j * 16, 16)] = jnp.zeros((16,), _f32)
        return 0
      return lax.fori_loop(0, D // 16, zfill_col, 0)
    lax.fori_loop(0, 8, zfill_row, 0)
    def zcopy(r, _):
      pltpu.sync_copy(rows_v.at[0, pl.ds(0, 8), :],
                      agg_sh.at[pl.ds(r0 + r * 8, 8), :])
      return 0
    lax.fori_loop(0, ROWS_PT // 8, zcopy, 0)
    if with_deg:
      pltpu.sync_copy(ones_hbm, ones_v)
      pltpu.sync_copy(zdeg_hbm, zvec)
      pltpu.sync_copy(zvec, deg_sh.at[pl.ds(r0, ROWS_PT)])

    plsc.subcore_barrier()

    def _scatter(k, b):
      pltpu.make_async_copy(table_hbm.at[src_t.at[k]], rows_v.at[b],
                            sem_g).wait()
      pltpu.sync_copy(rows_v.at[b], agg_sh.at[dst_t.at[k]], add=True)
      if with_deg:
        pltpu.sync_copy(ones_v, deg_sh.at[dst_t.at[k]], add=True)

    def _process(k, b, nb):
      pltpu.make_async_copy(table_hbm.at[src_t.at[k]], rows_v.at[b],
                            sem_g).wait()
      pltpu.async_copy(table_hbm.at[src_t.at[k + 1]], rows_v.at[nb], sem_g)
      pltpu.sync_copy(rows_v.at[b], agg_sh.at[dst_t.at[k]], add=True)
      if with_deg:
        pltpu.sync_copy(ones_v, deg_sh.at[dst_t.at[k]], add=True)

    for p in range(NCH // NPASS):
      pltpu.sync_copy(src_hbm.at[wid, pl.ds(p * NPASS, NPASS), :], src_t)
      pltpu.sync_copy(dst_hbm.at[wid, pl.ds(p * NPASS, NPASS), :], dst_t)
      pltpu.async_copy(table_hbm.at[src_t.at[0]], rows_v.at[0], sem_g)

      def step(j, _):
        _process(2 * j, 0, 1)
        _process(2 * j + 1, 1, 0)
        return 0
      lax.fori_loop(0, NPASS // 2 - 1, step, 0)
      _process(NPASS - 2, 0, 1)
      _scatter(NPASS - 1, 1)

    plsc.subcore_barrier()

    pltpu.sync_copy(agg_sh.at[pl.ds(r0, ROWS_PT), :],
                    agg_out.at[c, pl.ds(r0, ROWS_PT), :])
    if with_deg:
      pltpu.sync_copy(deg_sh.at[pl.ds(r0, ROWS_PT)], zvec)
      pltpu.sync_copy(zvec, deg_out.at[pl.ds(c * N_PAD + r0, ROWS_PT)])

  return pl.kernel(body, out_type=out_type, mesh=mesh, scratch_types=scratch)


_sc_agg_deg = _make_sc_agg(True)
_sc_agg = _make_sc_agg(False)

ROWS_B = 400
GRID = N_NODES // ROWS_B


def _tc1_body(p_ref, d_ref, w_ref, b_ref, o_ref):
  a = p_ref[0] + p_ref[1]
  d = jnp.maximum(d_ref[0] + d_ref[1], 1.0)
  a = a / d
  h = jnp.dot(a, w_ref[...], preferred_element_type=_f32) + b_ref[...]
  o_ref[...] = jnp.maximum(h, 0.0)


_tc1 = pl.pallas_call(
    _tc1_body,
    grid=(GRID,),
    in_specs=[
        pl.BlockSpec((NC, ROWS_B, D), lambda j: (0, j, 0)),
        pl.BlockSpec((NC, ROWS_B, 1), lambda j: (0, j, 0)),
        pl.BlockSpec((D, D), lambda j: (0, 0)),
        pl.BlockSpec((1, D), lambda j: (0, 0)),
    ],
    out_specs=pl.BlockSpec((ROWS_B, D), lambda j: (j, 0)),
    out_shape=jax.ShapeDtypeStruct((N_NODES, D), _f32),
)


def _tc2_body(p_ref, d_ref, w2_ref, b2_ref, wd_ref, bd_ref, o_ref):
  a = p_ref[0] + p_ref[1]
  d = jnp.maximum(d_ref[0] + d_ref[1], 1.0)
  a = a / d
  h = jnp.maximum(
      jnp.dot(a, w2_ref[...], preferred_element_type=_f32) + b2_ref[...], 0.0)
  lg = jnp.dot(h, wd_ref[...], preferred_element_type=_f32) + bd_ref[...]
  m = jnp.max(lg, axis=-1, keepdims=True)
  e = jnp.exp(lg - m)
  o_ref[...] = e / jnp.sum(e, axis=-1, keepdims=True)


_tc2 = pl.pallas_call(
    _tc2_body,
    grid=(GRID,),
    in_specs=[
        pl.BlockSpec((NC, ROWS_B, D), lambda j: (0, j, 0)),
        pl.BlockSpec((NC, ROWS_B, 1), lambda j: (0, j, 0)),
        pl.BlockSpec((D, D), lambda j: (0, 0)),
        pl.BlockSpec((1, D), lambda j: (0, 0)),
        pl.BlockSpec((D, NUM_CLASSES), lambda j: (0, 0)),
        pl.BlockSpec((1, NUM_CLASSES), lambda j: (0, 0)),
    ],
    out_specs=pl.BlockSpec((ROWS_B, NUM_CLASSES), lambda j: (j, 0)),
    out_shape=jax.ShapeDtypeStruct((N_NODES, NUM_CLASSES), _f32),
)


def kernel(x, edge_index, W1, b1, W2, b2, Wd, bd):
  src = edge_index[0].reshape(NW, NCH, CHUNK)
  dst = edge_index[1].reshape(NW, NCH, CHUNK)
  zdeg = jnp.zeros((ROWS_PT,), _f32)
  ones1 = jnp.ones((CHUNK,), _f32)
  agg1p, degf = _sc_agg_deg(x, src, dst, zdeg, ones1)
  degp = degf.reshape(NC, N_PAD, 1)
  h1 = _tc1(agg1p, degp, W1, b1.reshape(1, D))
  agg2p, = _sc_agg(h1, src, dst)
  out = _tc2(agg2p, degp, W2, b2.reshape(1, D), Wd, bd.reshape(1, NUM_CLASSES))
  return out


# speedup vs baseline: 1.9512x; 1.0009x over previous
import jax
import jax.numpy as jnp
from jax import lax
from jax.experimental import pallas as pl
from jax.experimental.pallas import tpu as pltpu
from jax.experimental.pallas import tpu_sc as plsc

N_NODES = 10000
N_PAD = 10112
N_EDGES = 320000
D = 128
NUM_CLASSES = 64

NC = 2
NS = 16
NW = NC * NS
EPT = N_EDGES // NW
CHUNK = 125
NCH = EPT // CHUNK
NPASS = NCH // 2
ROWS_PT = N_PAD // NS

_f32 = jnp.float32


def _make_sc_agg(with_deg):
  mesh = plsc.VectorSubcoreMesh(core_axis_name="c", subcore_axis_name="s")

  out_type = [jax.ShapeDtypeStruct((NC, N_PAD, D), _f32)]
  scratch = [
      pltpu.VMEM((NPASS, CHUNK), jnp.int32),
      pltpu.VMEM((NPASS, CHUNK), jnp.int32),
      pltpu.VMEM((2, CHUNK, D), _f32),
      pltpu.VMEM_SHARED((N_PAD, D), _f32),
      pltpu.SemaphoreType.DMA,
      pltpu.SemaphoreType.DMA,
  ]
  if with_deg:
    out_type.append(jax.ShapeDtypeStruct((NC * N_PAD,), _f32))
    scratch += [
        pltpu.VMEM((CHUNK,), _f32),
        pltpu.VMEM((ROWS_PT,), _f32),
        pltpu.VMEM_SHARED((N_PAD,), _f32),
    ]

  def body(table_hbm, src_hbm, dst_hbm, *rest):
    if with_deg:
      (zdeg_hbm, ones_hbm, agg_out, deg_out, src_t, dst_t, rows_v,
       agg_sh, sem_g, sem_s, ones_v, zvec, deg_sh) = rest
    else:
      agg_out, src_t, dst_t, rows_v, agg_sh, sem_g, sem_s = rest
    c = lax.axis_index("c")
    s = lax.axis_index("s")
    wid = s * NC + c
    r0 = s * ROWS_PT

    def zfill_row(i, _):
      def zfill_col(j, _):
        rows_v[0, i, pl.ds(j * 16, 16)] = jnp.zeros((16,), _f32)
        return 0
      return lax.fori_loop(0, D // 16, zfill_col, 0)
    lax.fori_loop(0, 8, zfill_row, 0)
    def zcopy(r, _):
      pltpu.sync_copy(rows_v.at[0, pl.ds(0, 8), :],
                      agg_sh.at[pl.ds(r0 + r * 8, 8), :])
      return 0
    lax.fori_loop(0, ROWS_PT // 8, zcopy, 0)
    if with_deg:
      pltpu.sync_copy(ones_hbm, ones_v)
      pltpu.sync_copy(zdeg_hbm, zvec)
      pltpu.sync_copy(zvec, deg_sh.at[pl.ds(r0, ROWS_PT)])

    plsc.subcore_barrier()

    def _process(k, b, nb, first=False, last=False):
      pltpu.make_async_copy(table_hbm.at[src_t.at[k]], rows_v.at[b],
                            sem_g).wait()
      if not first:
        pltpu.make_async_copy(rows_v.at[nb], agg_sh.at[dst_t.at[k]],
                              sem_s).wait()
      if not last:
        pltpu.async_copy(table_hbm.at[src_t.at[k + 1]], rows_v.at[nb], sem_g)
      pltpu.async_copy(rows_v.at[b], agg_sh.at[dst_t.at[k]], sem_s, add=True)
      if with_deg:
        pltpu.sync_copy(ones_v, deg_sh.at[dst_t.at[k]], add=True)

    for p in range(NCH // NPASS):
      pltpu.sync_copy(src_hbm.at[wid, pl.ds(p * NPASS, NPASS), :], src_t)
      pltpu.sync_copy(dst_hbm.at[wid, pl.ds(p * NPASS, NPASS), :], dst_t)
      pltpu.async_copy(table_hbm.at[src_t.at[0]], rows_v.at[0], sem_g)

      _process(0, 0, 1, first=True)
      def step(j, _):
        _process(2 * j + 1, 1, 0)
        _process(2 * j + 2, 0, 1)
        return 0
      lax.fori_loop(0, NPASS // 2 - 1, step, 0)
      _process(NPASS - 1, 1, 0, last=True)
      pltpu.make_async_copy(rows_v.at[1], agg_sh.at[dst_t.at[NPASS - 1]],
                            sem_s).wait()

    plsc.subcore_barrier()

    pltpu.sync_copy(agg_sh.at[pl.ds(r0, ROWS_PT), :],
                    agg_out.at[c, pl.ds(r0, ROWS_PT), :])
    if with_deg:
      pltpu.sync_copy(deg_sh.at[pl.ds(r0, ROWS_PT)], zvec)
      pltpu.sync_copy(zvec, deg_out.at[pl.ds(c * N_PAD + r0, ROWS_PT)])

  return pl.kernel(body, out_type=out_type, mesh=mesh, scratch_types=scratch)


_sc_agg_deg = _make_sc_agg(True)
_sc_agg = _make_sc_agg(False)

ROWS_B = 400
GRID = N_NODES // ROWS_B


def _tc1_body(p_ref, d_ref, w_ref, b_ref, o_ref):
  a = p_ref[0] + p_ref[1]
  d = jnp.maximum(d_ref[0] + d_ref[1], 1.0)
  a = a / d
  h = jnp.dot(a, w_ref[...], preferred_element_type=_f32) + b_ref[...]
  o_ref[...] = jnp.maximum(h, 0.0)


_tc1 = pl.pallas_call(
    _tc1_body,
    grid=(GRID,),
    in_specs=[
        pl.BlockSpec((NC, ROWS_B, D), lambda j: (0, j, 0)),
        pl.BlockSpec((NC, ROWS_B, 1), lambda j: (0, j, 0)),
        pl.BlockSpec((D, D), lambda j: (0, 0)),
        pl.BlockSpec((1, D), lambda j: (0, 0)),
    ],
    out_specs=pl.BlockSpec((ROWS_B, D), lambda j: (j, 0)),
    out_shape=jax.ShapeDtypeStruct((N_NODES, D), _f32),
)


def _tc2_body(p_ref, d_ref, w2_ref, b2_ref, wd_ref, bd_ref, o_ref):
  a = p_ref[0] + p_ref[1]
  d = jnp.maximum(d_ref[0] + d_ref[1], 1.0)
  a = a / d
  h = jnp.maximum(
      jnp.dot(a, w2_ref[...], preferred_element_type=_f32) + b2_ref[...], 0.0)
  lg = jnp.dot(h, wd_ref[...], preferred_element_type=_f32) + bd_ref[...]
  m = jnp.max(lg, axis=-1, keepdims=True)
  e = jnp.exp(lg - m)
  o_ref[...] = e / jnp.sum(e, axis=-1, keepdims=True)


_tc2 = pl.pallas_call(
    _tc2_body,
    grid=(GRID,),
    in_specs=[
        pl.BlockSpec((NC, ROWS_B, D), lambda j: (0, j, 0)),
        pl.BlockSpec((NC, ROWS_B, 1), lambda j: (0, j, 0)),
        pl.BlockSpec((D, D), lambda j: (0, 0)),
        pl.BlockSpec((1, D), lambda j: (0, 0)),
        pl.BlockSpec((D, NUM_CLASSES), lambda j: (0, 0)),
        pl.BlockSpec((1, NUM_CLASSES), lambda j: (0, 0)),
    ],
    out_specs=pl.BlockSpec((ROWS_B, NUM_CLASSES), lambda j: (j, 0)),
    out_shape=jax.ShapeDtypeStruct((N_NODES, NUM_CLASSES), _f32),
)


def kernel(x, edge_index, W1, b1, W2, b2, Wd, bd):
  src = edge_index[0].reshape(NW, NCH, CHUNK)
  dst = edge_index[1].reshape(NW, NCH, CHUNK)
  zdeg = jnp.zeros((ROWS_PT,), _f32)
  ones1 = jnp.ones((CHUNK,), _f32)
  agg1p, degf = _sc_agg_deg(x, src, dst, zdeg, ones1)
  degp = degf.reshape(NC, N_PAD, 1)
  h1 = _tc1(agg1p, degp, W1, b1.reshape(1, D))
  agg2p, = _sc_agg(h1, src, dst)
  out = _tc2(agg2p, degp, W2, b2.reshape(1, D), Wd, bd.reshape(1, NUM_CLASSES))
  return out


# speedup vs baseline: 1.9561x; 1.0025x over previous
import jax
import jax.numpy as jnp
from jax import lax
from jax.experimental import pallas as pl
from jax.experimental.pallas import tpu as pltpu
from jax.experimental.pallas import tpu_sc as plsc

N_NODES = 10000
N_PAD = 10112
N_EDGES = 320000
D = 128
NUM_CLASSES = 64

NC = 2
NS = 16
NW = NC * NS
EPT = N_EDGES // NW
CHUNK = 125
NCH = EPT // CHUNK
NPASS = NCH // 2
ROWS_PT = N_PAD // NS

_f32 = jnp.float32


def _make_sc_agg(with_deg):
  mesh = plsc.VectorSubcoreMesh(core_axis_name="c", subcore_axis_name="s")

  out_type = [jax.ShapeDtypeStruct((NC, N_PAD, D), _f32)]
  scratch = [
      pltpu.VMEM((NPASS, CHUNK), jnp.int32),
      pltpu.VMEM((NPASS, CHUNK), jnp.int32),
      pltpu.VMEM((2, CHUNK, D), _f32),
      pltpu.VMEM_SHARED((N_PAD, D), _f32),
      pltpu.SemaphoreType.DMA,
  ]
  if with_deg:
    out_type.append(jax.ShapeDtypeStruct((NC * N_PAD,), _f32))
    scratch += [
        pltpu.VMEM((CHUNK,), _f32),
        pltpu.VMEM((ROWS_PT,), _f32),
        pltpu.VMEM_SHARED((N_PAD,), _f32),
    ]

  def body(table_hbm, src_hbm, dst_hbm, *rest):
    if with_deg:
      (zdeg_hbm, ones_hbm, agg_out, deg_out, src_t, dst_t, rows_v,
       agg_sh, sem_g, ones_v, zvec, deg_sh) = rest
    else:
      agg_out, src_t, dst_t, rows_v, agg_sh, sem_g = rest
    c = lax.axis_index("c")
    s = lax.axis_index("s")
    wid = s * NC + c
    r0 = s * ROWS_PT

    def zfill_row(i, _):
      def zfill_col(j, _):
        rows_v[0, i, pl.ds(j * 16, 16)] = jnp.zeros((16,), _f32)
        return 0
      return lax.fori_loop(0, D // 16, zfill_col, 0)
    lax.fori_loop(0, 8, zfill_row, 0)
    def zcopy(r, _):
      pltpu.sync_copy(rows_v.at[0, pl.ds(0, 8), :],
                      agg_sh.at[pl.ds(r0 + r * 8, 8), :])
      return 0
    lax.fori_loop(0, ROWS_PT // 8, zcopy, 0)
    if with_deg:
      pltpu.sync_copy(ones_hbm, ones_v)
      pltpu.sync_copy(zdeg_hbm, zvec)
      pltpu.sync_copy(zvec, deg_sh.at[pl.ds(r0, ROWS_PT)])

    plsc.subcore_barrier()

    def _scatter(k, b):
      pltpu.make_async_copy(table_hbm.at[src_t.at[k]], rows_v.at[b],
                            sem_g).wait()
      pltpu.sync_copy(rows_v.at[b], agg_sh.at[dst_t.at[k]], add=True)
      if with_deg:
        pltpu.sync_copy(ones_v, deg_sh.at[dst_t.at[k]], add=True)

    def _process(k, b, nb):
      pltpu.make_async_copy(table_hbm.at[src_t.at[k]], rows_v.at[b],
                            sem_g).wait()
      pltpu.async_copy(table_hbm.at[src_t.at[k + 1]], rows_v.at[nb], sem_g)
      pltpu.sync_copy(rows_v.at[b], agg_sh.at[dst_t.at[k]], add=True)
      if with_deg:
        pltpu.sync_copy(ones_v, deg_sh.at[dst_t.at[k]], add=True)

    for p in range(NCH // NPASS):
      pltpu.sync_copy(src_hbm.at[wid, pl.ds(p * NPASS, NPASS), :], src_t)
      pltpu.sync_copy(dst_hbm.at[wid, pl.ds(p * NPASS, NPASS), :], dst_t)
      pltpu.async_copy(table_hbm.at[src_t.at[0]], rows_v.at[0], sem_g)

      def step(j, _):
        _process(2 * j, 0, 1)
        _process(2 * j + 1, 1, 0)
        return 0
      lax.fori_loop(0, NPASS // 2 - 1, step, 0)
      _process(NPASS - 2, 0, 1)
      _scatter(NPASS - 1, 1)

    plsc.subcore_barrier()

    pltpu.sync_copy(agg_sh.at[pl.ds(r0, ROWS_PT), :],
                    agg_out.at[c, pl.ds(r0, ROWS_PT), :])
    if with_deg:
      pltpu.sync_copy(deg_sh.at[pl.ds(r0, ROWS_PT)], zvec)
      pltpu.sync_copy(zvec, deg_out.at[pl.ds(c * N_PAD + r0, ROWS_PT)])

  return pl.kernel(body, out_type=out_type, mesh=mesh, scratch_types=scratch)


_sc_agg_deg = _make_sc_agg(True)
_sc_agg = _make_sc_agg(False)

ROWS_B = 400
GRID = N_NODES // ROWS_B


def _tc1_body(p_ref, d_ref, w_ref, b_ref, o_ref):
  a = p_ref[0] + p_ref[1]
  d = jnp.maximum(d_ref[0] + d_ref[1], 1.0)
  a = a / d
  h = jnp.dot(a, w_ref[...], preferred_element_type=_f32) + b_ref[...]
  o_ref[...] = jnp.maximum(h, 0.0)


_tc1 = pl.pallas_call(
    _tc1_body,
    grid=(GRID,),
    in_specs=[
        pl.BlockSpec((NC, ROWS_B, D), lambda j: (0, j, 0)),
        pl.BlockSpec((NC, ROWS_B, 1), lambda j: (0, j, 0)),
        pl.BlockSpec((D, D), lambda j: (0, 0)),
        pl.BlockSpec((1, D), lambda j: (0, 0)),
    ],
    out_specs=pl.BlockSpec((ROWS_B, D), lambda j: (j, 0)),
    out_shape=jax.ShapeDtypeStruct((N_NODES, D), _f32),
)


def _tc2_body(p_ref, d_ref, w2_ref, b2_ref, wd_ref, bd_ref, o_ref):
  a = p_ref[0] + p_ref[1]
  d = jnp.maximum(d_ref[0] + d_ref[1], 1.0)
  a = a / d
  h = jnp.maximum(
      jnp.dot(a, w2_ref[...], preferred_element_type=_f32) + b2_ref[...], 0.0)
  lg = jnp.dot(h, wd_ref[...], preferred_element_type=_f32) + bd_ref[...]
  m = jnp.max(lg, axis=-1, keepdims=True)
  e = jnp.exp(lg - m)
  o_ref[...] = e / jnp.sum(e, axis=-1, keepdims=True)


_tc2 = pl.pallas_call(
    _tc2_body,
    grid=(GRID,),
    in_specs=[
        pl.BlockSpec((NC, ROWS_B, D), lambda j: (0, j, 0)),
        pl.BlockSpec((NC, ROWS_B, 1), lambda j: (0, j, 0)),
        pl.BlockSpec((D, D), lambda j: (0, 0)),
        pl.BlockSpec((1, D), lambda j: (0, 0)),
        pl.BlockSpec((D, NUM_CLASSES), lambda j: (0, 0)),
        pl.BlockSpec((1, NUM_CLASSES), lambda j: (0, 0)),
    ],
    out_specs=pl.BlockSpec((ROWS_B, NUM_CLASSES), lambda j: (j, 0)),
    out_shape=jax.ShapeDtypeStruct((N_NODES, NUM_CLASSES), _f32),
)


def kernel(x, edge_index, W1, b1, W2, b2, Wd, bd):
  src = edge_index[0].reshape(NW, NCH, CHUNK)
  dst = edge_index[1].reshape(NW, NCH, CHUNK)
  zdeg = jnp.zeros((ROWS_PT,), _f32)
  ones1 = jnp.ones((CHUNK,), _f32)
  agg1p, degf = _sc_agg_deg(x, src, dst, zdeg, ones1)
  degp = degf.reshape(NC, N_PAD, 1)
  h1 = _tc1(agg1p, degp, W1, b1.reshape(1, D))
  agg2p, = _sc_agg(h1, src, dst)
  out = _tc2(agg2p, degp, W2, b2.reshape(1, D), Wd, bd.reshape(1, NUM_CLASSES))
  return out
